# baseline (speedup 1.0000x reference)
import functools

import jax
import jax.numpy as jnp
from jax import lax
from jax.experimental import pallas as pl
from jax.experimental.pallas import tpu as pltpu
from jax.experimental.pallas import tpu_sc as plsc

NC = 2
NS = 16
NW = NC * NS

CHUNK = 1600


def _sc_gather(table, ids, n):
    d = table.shape[1]
    per_w = n // NW
    steps = per_w // CHUNK
    mesh = plsc.VectorSubcoreMesh(core_axis_name="c", subcore_axis_name="s")

    @functools.partial(
        pl.kernel,
        out_type=jax.ShapeDtypeStruct((n, d), jnp.float32),
        mesh=mesh,
        scratch_types=[
            pltpu.VMEM((CHUNK,), jnp.int32),
            pltpu.VMEM((CHUNK, d), jnp.float32),
            pltpu.SemaphoreType.DMA,
        ],
        compiler_params=pltpu.CompilerParams(use_tc_tiling_on_sc=False),
    )
    def gather_kernel(table_hbm, idx_hbm, out_hbm, idx_v, rows_v, sem):
        wid = lax.axis_index("s") * NC + lax.axis_index("c")
        base = wid * per_w

        def body(i, carry):
            off = base + i * CHUNK
            pltpu.sync_copy(idx_hbm.at[pl.ds(off, CHUNK)], idx_v)
            pltpu.async_copy(table_hbm.at[idx_v], rows_v, sem).wait()
            pltpu.sync_copy(rows_v, out_hbm.at[pl.ds(off, CHUNK)])
            return carry

        lax.fori_loop(0, steps, body, 0)

    return gather_kernel(table, ids)


def _tc_project(x, w, b, block_n):
    n, d = x.shape
    h = w.shape[1]

    def matmul_kernel(x_ref, w_ref, b_ref, o_ref):
        o_ref[...] = (
            jnp.dot(x_ref[...], w_ref[...], preferred_element_type=jnp.float32)
            + b_ref[...]
        )

    return pl.pallas_call(
        matmul_kernel,
        grid=(n // block_n,),
        in_specs=[
            pl.BlockSpec((block_n, d), lambda i: (i, 0)),
            pl.BlockSpec((d, h), lambda i: (0, 0)),
            pl.BlockSpec((1, h), lambda i: (0, 0)),
        ],
        out_specs=pl.BlockSpec((block_n, h), lambda i: (i, 0)),
        out_shape=jax.ShapeDtypeStruct((n, h), jnp.float32),
    )(x, w, b.reshape(1, h))


def kernel(input_ids, token_embed, W, b):
    bsz, seq = input_ids.shape
    n = bsz * seq
    h = W.shape[1]
    ids = input_ids.reshape(n).astype(jnp.int32)
    rows = _sc_gather(token_embed, ids, n)
    out = _tc_project(rows, W, b, block_n=2048)
    return out.reshape(bsz, seq, h)


# speedup vs baseline: 1.1108x; 1.1108x over previous
import functools

import jax
import jax.numpy as jnp
from jax import lax
from jax.experimental import pallas as pl
from jax.experimental.pallas import tpu as pltpu
from jax.experimental.pallas import tpu_sc as plsc

NC = 2
NS = 16
NW = NC * NS

CHUNK = 800


def _sc_gather_packed(table, ids, n):
    d = table.shape[1]
    np_ = n // 2
    per_w = np_ // NW
    steps = per_w // CHUNK
    mesh = plsc.VectorSubcoreMesh(core_axis_name="c", subcore_axis_name="s")

    @functools.partial(
        pl.kernel,
        out_type=jax.ShapeDtypeStruct((np_, 2 * d), jnp.float32),
        mesh=mesh,
        scratch_types=[
            pltpu.VMEM((CHUNK,), jnp.int32),
            pltpu.VMEM((CHUNK, d), jnp.float32),
            pltpu.SemaphoreType.DMA,
        ],
        compiler_params=pltpu.CompilerParams(use_tc_tiling_on_sc=False),
    )
    def gather_kernel(table_hbm, idx_hbm, out_hbm, idx_v, rows_v, sem):
        wid = lax.axis_index("s") * NC + lax.axis_index("c")
        base = wid * per_w

        def body(i, carry):
            r0 = base + i * CHUNK
            pltpu.sync_copy(idx_hbm.at[pl.ds(r0, CHUNK)], idx_v)
            pltpu.async_copy(table_hbm.at[idx_v], rows_v, sem).wait()
            pltpu.sync_copy(rows_v, out_hbm.at[pl.ds(r0, CHUNK), pl.ds(0, d)])
            pltpu.sync_copy(idx_hbm.at[pl.ds(np_ + r0, CHUNK)], idx_v)
            pltpu.async_copy(table_hbm.at[idx_v], rows_v, sem).wait()
            pltpu.sync_copy(rows_v, out_hbm.at[pl.ds(r0, CHUNK), pl.ds(d, d)])
            return carry

        lax.fori_loop(0, steps, body, 0)

    return gather_kernel(table, ids)


def _tc_project_pair(x2, wa, wb, b, block_n):
    np_, k = x2.shape
    h = wa.shape[1]

    def matmul_kernel(x_ref, wa_ref, wb_ref, b_ref, o_ref):
        x = x_ref[...]
        o_ref[0] = jnp.dot(x, wa_ref[...], preferred_element_type=jnp.float32) + b_ref[...]
        o_ref[1] = jnp.dot(x, wb_ref[...], preferred_element_type=jnp.float32) + b_ref[...]

    return pl.pallas_call(
        matmul_kernel,
        grid=(np_ // block_n,),
        in_specs=[
            pl.BlockSpec((block_n, k), lambda i: (i, 0)),
            pl.BlockSpec((k, h), lambda i: (0, 0)),
            pl.BlockSpec((k, h), lambda i: (0, 0)),
            pl.BlockSpec((1, h), lambda i: (0, 0)),
        ],
        out_specs=pl.BlockSpec((2, block_n, h), lambda i: (0, i, 0)),
        out_shape=jax.ShapeDtypeStruct((2, np_, h), jnp.float32),
    )(x2, wa, wb, b.reshape(1, h))


def kernel(input_ids, token_embed, W, b):
    bsz, seq = input_ids.shape
    n = bsz * seq
    d, h = W.shape
    ids = input_ids.reshape(n).astype(jnp.int32)
    x2 = _sc_gather_packed(token_embed, ids, n)
    zeros = jnp.zeros_like(W)
    wa = jnp.concatenate([W, zeros], axis=0)
    wb = jnp.concatenate([zeros, W], axis=0)
    out3 = _tc_project_pair(x2, wa, wb, b, block_n=1024)
    return out3.reshape(bsz, seq, h)


# speedup vs baseline: 1.1414x; 1.0276x over previous
import functools

import jax
import jax.numpy as jnp
from jax import lax
from jax.experimental import pallas as pl
from jax.experimental.pallas import tpu as pltpu
from jax.experimental.pallas import tpu_sc as plsc

NC = 2
NS = 16
NW = NC * NS

CHUNK = 800


def _sc_gather_packed(table, ids, n):
    d = table.shape[1]
    np_ = n // 2
    per_w = np_ // NW
    steps = per_w // CHUNK
    mesh = plsc.VectorSubcoreMesh(core_axis_name="c", subcore_axis_name="s")

    @functools.partial(
        pl.kernel,
        out_type=jax.ShapeDtypeStruct((np_, 2 * d), jnp.float32),
        mesh=mesh,
        scratch_types=[
            pltpu.VMEM((CHUNK,), jnp.int32),
            pltpu.VMEM((CHUNK, d), jnp.float32),
            pltpu.SemaphoreType.DMA,
        ],
        compiler_params=pltpu.CompilerParams(use_tc_tiling_on_sc=False),
    )
    def gather_kernel(table_hbm, idx_hbm, out_hbm, idx_v, rows_v, sem):
        wid = lax.axis_index("s") * NC + lax.axis_index("c")
        base = wid * per_w

        def body(i, carry):
            r0 = base + i * CHUNK
            pltpu.sync_copy(idx_hbm.at[pl.ds(r0, CHUNK)], idx_v)
            pltpu.async_copy(table_hbm.at[idx_v], rows_v, sem).wait()
            pltpu.sync_copy(rows_v, out_hbm.at[pl.ds(r0, CHUNK), pl.ds(0, d)])
            pltpu.sync_copy(idx_hbm.at[pl.ds(np_ + r0, CHUNK)], idx_v)
            pltpu.async_copy(table_hbm.at[idx_v], rows_v, sem).wait()
            pltpu.sync_copy(rows_v, out_hbm.at[pl.ds(r0, CHUNK), pl.ds(d, d)])
            return carry

        lax.fori_loop(0, steps, body, 0)

    return gather_kernel(table, ids)


def _tc_project_pair(x2, wa, wb, b, bsz, seq, batch_block):
    np_, k = x2.shape
    h = wa.shape[1]
    rows_per_block = batch_block * seq

    def matmul_kernel(x_ref, wa_ref, wb_ref, b_ref, o_ref):
        x = x_ref[...]
        y0 = jnp.dot(x, wa_ref[...], preferred_element_type=jnp.float32) + b_ref[...]
        y1 = jnp.dot(x, wb_ref[...], preferred_element_type=jnp.float32) + b_ref[...]
        o_ref[0] = y0.reshape(batch_block, seq, h)
        o_ref[1] = y1.reshape(batch_block, seq, h)

    return pl.pallas_call(
        matmul_kernel,
        grid=(np_ // rows_per_block,),
        in_specs=[
            pl.BlockSpec((rows_per_block, k), lambda i: (i, 0)),
            pl.BlockSpec((k, h), lambda i: (0, 0)),
            pl.BlockSpec((k, h), lambda i: (0, 0)),
            pl.BlockSpec((1, h), lambda i: (0, 0)),
        ],
        out_specs=pl.BlockSpec(
            (2, batch_block, seq, h), lambda i: (0, i, 0, 0)
        ),
        out_shape=jax.ShapeDtypeStruct((2, bsz // 2, seq, h), jnp.float32),
    )(x2, wa, wb, b.reshape(1, h))


def kernel(input_ids, token_embed, W, b):
    bsz, seq = input_ids.shape
    n = bsz * seq
    d, h = W.shape
    ids = input_ids.reshape(n).astype(jnp.int32)
    x2 = _sc_gather_packed(token_embed, ids, n)
    zeros = jnp.zeros_like(W)
    wa = jnp.concatenate([W, zeros], axis=0)
    wb = jnp.concatenate([zeros, W], axis=0)
    out4 = _tc_project_pair(x2, wa, wb, b, bsz, seq, batch_block=8)
    return out4.reshape(bsz, seq, h)
